# baseline (speedup 1.0000x reference)
import functools

import jax
import jax.numpy as jnp
from jax import lax
from jax.experimental import pallas as pl
from jax.experimental.pallas import tpu as pltpu
from jax.experimental.pallas import tpu_sc as plsc

_B = 4096
_D = 32
_MEAN = 3.5
_NC = 2
_NS = 16
_NW = _NC * _NS
_BPW = _B // _NW
_L = 16


def _sc_body(user_hbm, item_hbm, uemb_hbm, iemb_hbm, ubias_hbm, ibias_hbm,
             a_out, d_out,
             uidx_v, iidx_v, urows_v, irows_v, ub_v, ib_v, a_loc, d_loc, sem):
    wid = lax.axis_index("s") * _NC + lax.axis_index("c")
    base = wid * _BPW

    pltpu.sync_copy(user_hbm.at[pl.ds(base, _BPW)], uidx_v)
    pltpu.sync_copy(item_hbm.at[pl.ds(base, _BPW)], iidx_v)

    copies = [
        pltpu.async_copy(uemb_hbm.at[uidx_v], urows_v, sem),
        pltpu.async_copy(iemb_hbm.at[iidx_v], irows_v, sem),
        pltpu.async_copy(ubias_hbm.at[uidx_v], ub_v, sem),
        pltpu.async_copy(ibias_hbm.at[iidx_v], ib_v, sem),
    ]
    for cp in copies:
        cp.wait()

    lane = lax.iota(jnp.int32, _L)
    for g in range(_BPW // _L):
        acc = jnp.zeros((_L,), jnp.float32)
        for j in range(_L):
            r = g * _L + j
            u0 = urows_v[r, pl.ds(0, _L)]
            u1 = urows_v[r, pl.ds(_L, _L)]
            v0 = irows_v[r, pl.ds(0, _L)]
            v1 = irows_v[r, pl.ds(_L, _L)]
            s = jnp.sum(u0 * v0 + u1 * v1)
            acc = jnp.where(lane == j, s, acc)
        sl = pl.ds(g * _L, _L)
        d_loc[sl] = acc
        a_loc[sl] = ub_v[sl] + ib_v[sl]

    pltpu.sync_copy(a_loc, a_out.at[pl.ds(base, _BPW)])
    pltpu.sync_copy(d_loc, d_out.at[pl.ds(base, _BPW)])


_sc_gather = pl.kernel(
    _sc_body,
    out_type=(jax.ShapeDtypeStruct((_B,), jnp.float32),
              jax.ShapeDtypeStruct((_B,), jnp.float32)),
    mesh=plsc.VectorSubcoreMesh(core_axis_name="c", subcore_axis_name="s"),
    compiler_params=pltpu.CompilerParams(needs_layout_passes=False,
                                         use_tc_tiling_on_sc=False),
    scratch_types=[
        pltpu.VMEM((_BPW,), jnp.int32),
        pltpu.VMEM((_BPW,), jnp.int32),
        pltpu.VMEM((_BPW, _D), jnp.float32),
        pltpu.VMEM((_BPW, _D), jnp.float32),
        pltpu.VMEM((_BPW,), jnp.float32),
        pltpu.VMEM((_BPW,), jnp.float32),
        pltpu.VMEM((_BPW,), jnp.float32),
        pltpu.VMEM((_BPW,), jnp.float32),
        pltpu.SemaphoreType.DMA,
    ],
)

_ROWS = 512


def _bcast_body(a_ref, d_ref, o_ref):
    o_ref[...] = a_ref[...] + d_ref[...] + _MEAN


_bcast = pl.pallas_call(
    _bcast_body,
    grid=(_B // _ROWS,),
    in_specs=[
        pl.BlockSpec((_ROWS, 1), lambda i: (i, 0)),
        pl.BlockSpec((1, _B), lambda i: (0, 0)),
    ],
    out_specs=pl.BlockSpec((_ROWS, _B), lambda i: (i, 0)),
    out_shape=jax.ShapeDtypeStruct((_B, _B), jnp.float32),
)


def kernel(user, item, user_embeddings, item_embeddings, user_biases, item_biases):
    user = user.astype(jnp.int32)
    item = item.astype(jnp.int32)
    ub1 = user_biases.reshape(-1)
    ib1 = item_biases.reshape(-1)
    a, d = _sc_gather(user, item, user_embeddings, item_embeddings, ub1, ib1)
    return _bcast(a.reshape(_B, 1), d.reshape(1, _B))
